# baseline (speedup 1.0000x reference)
import functools

import jax
import jax.numpy as jnp
from jax import lax
from jax.experimental import pallas as pl
from jax.experimental.pallas import tpu as pltpu
from jax.experimental.pallas import tpu_sc as plsc

_B, _L, _V = 16, 50, 100000
_N = _B * _L
_NPAD = 1024
_CHUNK = 128
_NCHUNKS = _NPAD // _CHUNK


def _sc_loss(flat_inp, tgt_pad, msk_pad):
    mesh = plsc.VectorSubcoreMesh(core_axis_name="c", subcore_axis_name="s")

    @functools.partial(
        pl.kernel,
        mesh=mesh,
        out_type=jax.ShapeDtypeStruct((16,), jnp.float32),
        compiler_params=pltpu.CompilerParams(needs_layout_passes=False),
        scratch_types=[
            pltpu.VMEM((_NPAD,), jnp.int32),
            pltpu.VMEM((_NPAD,), jnp.float32),
            pltpu.VMEM((_NPAD,), jnp.int32),
            pltpu.VMEM((_NPAD,), jnp.float32),
            pltpu.VMEM((16,), jnp.float32),
            pltpu.SemaphoreType.DMA,
        ],
    )
    def k(inp_hbm, tgt_hbm, msk_hbm, out_hbm, tgt_v, msk_v, idx_v, got_v,
          out_v, sem):
        c = lax.axis_index("c")
        s = lax.axis_index("s")

        @pl.when(jnp.logical_and(c == 0, s == 0))
        def _():
            pltpu.sync_copy(tgt_hbm, tgt_v)
            pltpu.sync_copy(msk_hbm, msk_v)
            for j in range(_NPAD // 16):
                row = jnp.minimum(lax.iota(jnp.int32, 16) + (j * 16), _N - 1)
                idx_v[pl.ds(j * 16, 16)] = row * _V + tgt_v[pl.ds(j * 16, 16)]
            cps = []
            for t in range(_NCHUNKS):
                cps.append(pltpu.async_copy(
                    inp_hbm.at[idx_v.at[pl.ds(t * _CHUNK, _CHUNK)]],
                    got_v.at[pl.ds(t * _CHUNK, _CHUNK)], sem))
            for cp in cps:
                cp.wait()
            num = jnp.zeros((16,), jnp.float32)
            den = jnp.zeros((16,), jnp.float32)
            for j in range(_NPAD // 16):
                g = got_v[pl.ds(j * 16, 16)]
                m = msk_v[pl.ds(j * 16, 16)]
                num = num + g * m
                den = den + m
            def lane_sum(vec):
                for sh in (8, 4, 2, 1):
                    out_v[...] = vec
                    ix = jnp.bitwise_and(lax.iota(jnp.int32, 16) + sh, 15)
                    vec = vec + plsc.load_gather(out_v, [ix])
                return vec

            num_t = lane_sum(num)
            den_t = lane_sum(den)
            out_v[...] = -(num_t / den_t)
            pltpu.sync_copy(out_v, out_hbm)

    return k(flat_inp, tgt_pad, msk_pad)


def kernel(input, target, mask):
    L = input.shape[1]
    target = target[:, :L]
    mask = mask[:, :L]
    flat = input.reshape(-1)
    tgt = target.reshape(-1).astype(jnp.int32)
    msk = mask.reshape(-1).astype(jnp.float32)
    tgt_pad = jnp.zeros((_NPAD,), jnp.int32).at[:_N].set(tgt)
    msk_pad = jnp.zeros((_NPAD,), jnp.float32).at[:_N].set(msk)
    out = _sc_loss(flat, tgt_pad, msk_pad)
    return out[0]


# speedup vs baseline: 18.0176x; 18.0176x over previous
import functools

import jax
import jax.numpy as jnp
from jax import lax
from jax.experimental import pallas as pl
from jax.experimental.pallas import tpu as pltpu
from jax.experimental.pallas import tpu_sc as plsc

_B, _L, _V = 16, 50, 100000
_N = _B * _L
_NPAD = 1024
_NW = 16
_PW = _NPAD // _NW
_WIN = 16


def _sc_loss(inp, tgt_pad, msk_pad):
    mesh = plsc.VectorSubcoreMesh(core_axis_name="c", subcore_axis_name="s")

    @functools.partial(
        pl.kernel,
        mesh=mesh,
        out_type=jax.ShapeDtypeStruct((_NW + 1, 2, 16), jnp.float32),
        compiler_params=pltpu.CompilerParams(needs_layout_passes=False),
        scratch_types=[
            pltpu.VMEM((_PW,), jnp.int32),
            pltpu.VMEM((_PW,), jnp.float32),
            pltpu.VMEM((_PW, _WIN), jnp.float32),
            pltpu.VMEM((2, 16), jnp.float32),
            pltpu.VMEM((_NW, 2, 16), jnp.float32),
            pltpu.VMEM((16,), jnp.float32),
            pltpu.VMEM((16,), jnp.float32),
            pltpu.SemaphoreType.DMA,
        ],
    )
    def k(inp_hbm, tgt_hbm, msk_hbm, out_hbm, tgt_v, msk_v, win_v, part_v,
          red_v, bfly_v, fin_v, sem):
        c = lax.axis_index("c")
        s = lax.axis_index("s")

        @pl.when(c == 0)
        def _():
            base = pl.multiple_of(s * _PW, _PW)
            pltpu.sync_copy(tgt_hbm.at[pl.ds(base, _PW)], tgt_v)
            pltpu.sync_copy(msk_hbm.at[pl.ds(base, _PW)], msk_v)
            cps = []
            for i in range(_PW):
                if i % 16 == 0:
                    t16 = tgt_v[pl.ds(i, 16)]
                r = jnp.minimum(base + i, _N - 1)
                b = r // _L
                l = r % _L
                t = t16[i % 16]
                v0 = jnp.minimum((t // _WIN) * _WIN, _V - _WIN)
                cps.append(pltpu.async_copy(
                    inp_hbm.at[b, l, pl.ds(v0, _WIN)], win_v.at[i], sem))
                if len(cps) == 16:
                    for cp in cps:
                        cp.wait()
                    cps = []
            for cp in cps:
                cp.wait()
            num = jnp.zeros((16,), jnp.float32)
            den = jnp.zeros((16,), jnp.float32)
            for j in range(_PW // 16):
                t16 = tgt_v[pl.ds(j * 16, 16)]
                v016 = jnp.minimum((t16 // _WIN) * _WIN, _V - _WIN)
                off16 = t16 - v016
                rows16 = lax.iota(jnp.int32, 16) + (j * 16)
                g = plsc.load_gather(win_v, [rows16, off16])
                m = msk_v[pl.ds(j * 16, 16)]
                num = num + g * m
                den = den + m
            part_v[0] = num
            part_v[1] = den
            pltpu.sync_copy(part_v, out_hbm.at[s])

        plsc.subcore_barrier()

        @pl.when(jnp.logical_and(c == 0, s == 0))
        def _():
            pltpu.sync_copy(out_hbm.at[pl.ds(0, _NW)], red_v)
            num = jnp.zeros((16,), jnp.float32)
            den = jnp.zeros((16,), jnp.float32)
            for w in range(_NW):
                num = num + red_v[w, 0]
                den = den + red_v[w, 1]

            def lane_sum(vec):
                for sh in (8, 4, 2, 1):
                    bfly_v[...] = vec
                    ix = jnp.bitwise_and(lax.iota(jnp.int32, 16) + sh, 15)
                    vec = vec + plsc.load_gather(bfly_v, [ix])
                return vec

            num_t = lane_sum(num)
            den_t = lane_sum(den)
            fin_v[...] = -(num_t / den_t)
            pltpu.sync_copy(fin_v, out_hbm.at[_NW, 0])

    return k(inp, tgt_pad, msk_pad)


def kernel(input, target, mask):
    L = input.shape[1]
    target = target[:, :L]
    mask = mask[:, :L]
    tgt = target.reshape(-1).astype(jnp.int32)
    msk = mask.reshape(-1).astype(jnp.float32)
    tgt_pad = jnp.zeros((_NPAD,), jnp.int32).at[:_N].set(tgt)
    msk_pad = jnp.zeros((_NPAD,), jnp.float32).at[:_N].set(msk)
    out = _sc_loss(input, tgt_pad, msk_pad)
    return out[_NW, 0, 0]


# speedup vs baseline: 148.3458x; 8.2334x over previous
import functools

import jax
import jax.numpy as jnp
from jax import lax
from jax.experimental import pallas as pl
from jax.experimental.pallas import tpu as pltpu
from jax.experimental.pallas import tpu_sc as plsc

_B, _L, _V = 16, 50, 100000
_N = _B * _L
_NPAD = 1024
_NW = 16
_PW = _NPAD // _NW
_WIN = 16


def _sc_loss(inp, tgt_pad, msk_pad):
    mesh = plsc.VectorSubcoreMesh(core_axis_name="c", subcore_axis_name="s")

    @functools.partial(
        pl.kernel,
        mesh=mesh,
        out_type=jax.ShapeDtypeStruct((_NW + 1, 2, 16), jnp.float32),
        compiler_params=pltpu.CompilerParams(needs_layout_passes=False),
        scratch_types=[
            pltpu.VMEM((_PW,), jnp.int32),
            pltpu.VMEM((_PW,), jnp.float32),
            pltpu.VMEM((_PW, _WIN), jnp.float32),
            pltpu.VMEM((2, 16), jnp.float32),
            pltpu.VMEM((_NW, 2, 16), jnp.float32),
            pltpu.VMEM((16,), jnp.float32),
            pltpu.VMEM((16,), jnp.float32),
            pltpu.SemaphoreType.DMA,
        ],
    )
    def k(inp_hbm, tgt_hbm, msk_hbm, out_hbm, tgt_v, msk_v, win_v, part_v,
          red_v, bfly_v, fin_v, sem):
        c = lax.axis_index("c")
        s = lax.axis_index("s")

        @pl.when(c == 0)
        def _():
            base = pl.multiple_of(s * _PW, _PW)
            pltpu.sync_copy(tgt_hbm.at[pl.ds(base, _PW)], tgt_v)
            pltpu.sync_copy(msk_hbm.at[pl.ds(base, _PW)], msk_v)
            cps = []
            for i in range(_PW):
                if i % 16 == 0:
                    t16 = tgt_v[pl.ds(i, 16)]
                r = jnp.minimum(base + i, _N - 1)
                b = r // _L
                l = r % _L
                t = t16[i % 16]
                v0 = jnp.minimum((t // _WIN) * _WIN, _V - _WIN)
                cps.append(pltpu.async_copy(
                    inp_hbm.at[l, b, pl.ds(v0, _WIN)], win_v.at[i], sem))
                if len(cps) == 16:
                    for cp in cps:
                        cp.wait()
                    cps = []
            for cp in cps:
                cp.wait()
            num = jnp.zeros((16,), jnp.float32)
            den = jnp.zeros((16,), jnp.float32)
            for j in range(_PW // 16):
                t16 = tgt_v[pl.ds(j * 16, 16)]
                v016 = jnp.minimum((t16 // _WIN) * _WIN, _V - _WIN)
                off16 = t16 - v016
                rows16 = lax.iota(jnp.int32, 16) + (j * 16)
                g = plsc.load_gather(win_v, [rows16, off16])
                m = msk_v[pl.ds(j * 16, 16)]
                num = num + g * m
                den = den + m
            part_v[0] = num
            part_v[1] = den
            pltpu.sync_copy(part_v, out_hbm.at[s])

        plsc.subcore_barrier()

        @pl.when(jnp.logical_and(c == 0, s == 0))
        def _():
            pltpu.sync_copy(out_hbm.at[pl.ds(0, _NW)], red_v)
            num = jnp.zeros((16,), jnp.float32)
            den = jnp.zeros((16,), jnp.float32)
            for w in range(_NW):
                num = num + red_v[w, 0]
                den = den + red_v[w, 1]

            def lane_sum(vec):
                for sh in (8, 4, 2, 1):
                    bfly_v[...] = vec
                    ix = jnp.bitwise_and(lax.iota(jnp.int32, 16) + sh, 15)
                    vec = vec + plsc.load_gather(bfly_v, [ix])
                return vec

            num_t = lane_sum(num)
            den_t = lane_sum(den)
            fin_v[...] = -(num_t / den_t)
            pltpu.sync_copy(fin_v, out_hbm.at[_NW, 0])

    return k(inp, tgt_pad, msk_pad)


def kernel(input, target, mask):
    L = input.shape[1]
    target = target[:, :L]
    mask = mask[:, :L]
    tgt = target.reshape(-1).astype(jnp.int32)
    msk = mask.reshape(-1).astype(jnp.float32)
    tgt_pad = jnp.zeros((_NPAD,), jnp.int32).at[:_N].set(tgt)
    msk_pad = jnp.zeros((_NPAD,), jnp.float32).at[:_N].set(msk)
    inp_t = jnp.transpose(input, (1, 0, 2))
    out = _sc_loss(inp_t, tgt_pad, msk_pad)
    return out[_NW, 0, 0]


# speedup vs baseline: 157.9366x; 1.0647x over previous
import functools

import jax
import jax.numpy as jnp
from jax import lax
from jax.experimental import pallas as pl
from jax.experimental.pallas import tpu as pltpu
from jax.experimental.pallas import tpu_sc as plsc

_B, _L, _V = 16, 50, 100000
_N = _B * _L
_NPAD = 1024
_NW = 16
_PW = _NPAD // _NW
_WIN = 16


def _sc_loss(inp, tgt_pad, msk_pad):
    mesh = plsc.VectorSubcoreMesh(core_axis_name="c", subcore_axis_name="s")

    @functools.partial(
        pl.kernel,
        mesh=mesh,
        out_type=jax.ShapeDtypeStruct((_NW + 1, 2, 16), jnp.float32),
        compiler_params=pltpu.CompilerParams(needs_layout_passes=False),
        scratch_types=[
            pltpu.VMEM((_PW,), jnp.int32),
            pltpu.VMEM((_PW,), jnp.float32),
            pltpu.VMEM((_PW, _WIN), jnp.float32),
            pltpu.VMEM((2, 16), jnp.float32),
            pltpu.VMEM((_NW, 2, 16), jnp.float32),
            pltpu.VMEM((16,), jnp.float32),
            pltpu.VMEM((16,), jnp.float32),
            pltpu.SemaphoreType.DMA,
        ],
    )
    def k(inp_hbm, tgt_hbm, msk_hbm, out_hbm, tgt_v, msk_v, win_v, part_v,
          red_v, bfly_v, fin_v, sem):
        c = lax.axis_index("c")
        s = lax.axis_index("s")

        @pl.when(c == 0)
        def _():
            base = pl.multiple_of(s * _PW, _PW)
            cp_t = pltpu.async_copy(tgt_hbm.at[pl.ds(base, _PW)], tgt_v, sem)
            cp_m = pltpu.async_copy(msk_hbm.at[pl.ds(base, _PW)], msk_v, sem)
            cp_t.wait()
            cp_m.wait()
            cps = []
            for i in range(_PW):
                if i % 16 == 0:
                    t16 = tgt_v[pl.ds(i, 16)]
                r = jnp.minimum(base + i, _N - 1)
                b = r // _L
                l = r % _L
                t = t16[i % 16]
                v0 = jnp.minimum((t // _WIN) * _WIN, _V - _WIN)
                cps.append(pltpu.async_copy(
                    inp_hbm.at[l, b, pl.ds(v0, _WIN)], win_v.at[i], sem))
            for cp in cps:
                cp.wait()
            num = jnp.zeros((16,), jnp.float32)
            den = jnp.zeros((16,), jnp.float32)
            for j in range(_PW // 16):
                t16 = tgt_v[pl.ds(j * 16, 16)]
                v016 = jnp.minimum((t16 // _WIN) * _WIN, _V - _WIN)
                off16 = t16 - v016
                rows16 = lax.iota(jnp.int32, 16) + (j * 16)
                g = plsc.load_gather(win_v, [rows16, off16])
                m = msk_v[pl.ds(j * 16, 16)]
                num = num + g * m
                den = den + m
            part_v[0] = num
            part_v[1] = den
            pltpu.sync_copy(part_v, out_hbm.at[s])

        plsc.subcore_barrier()

        @pl.when(jnp.logical_and(c == 0, s == 0))
        def _():
            pltpu.sync_copy(out_hbm.at[pl.ds(0, _NW)], red_v)
            num = jnp.zeros((16,), jnp.float32)
            den = jnp.zeros((16,), jnp.float32)
            for w in range(_NW):
                num = num + red_v[w, 0]
                den = den + red_v[w, 1]

            def lane_sum(vec):
                for sh in (8, 4, 2, 1):
                    bfly_v[...] = vec
                    ix = jnp.bitwise_and(lax.iota(jnp.int32, 16) + sh, 15)
                    vec = vec + plsc.load_gather(bfly_v, [ix])
                return vec

            num_t = lane_sum(num)
            den_t = lane_sum(den)
            fin_v[...] = -(num_t / den_t)
            pltpu.sync_copy(fin_v, out_hbm.at[_NW, 0])

    return k(inp, tgt_pad, msk_pad)


def kernel(input, target, mask):
    L = input.shape[1]
    target = target[:, :L]
    mask = mask[:, :L]
    tgt = target.reshape(-1).astype(jnp.int32)
    msk = mask.reshape(-1).astype(jnp.float32)
    tgt_pad = jnp.zeros((_NPAD,), jnp.int32).at[:_N].set(tgt)
    msk_pad = jnp.zeros((_NPAD,), jnp.float32).at[:_N].set(msk)
    inp_t = jnp.transpose(input, (1, 0, 2))
    out = _sc_loss(inp_t, tgt_pad, msk_pad)
    return out[_NW, 0, 0]


# speedup vs baseline: 173.9219x; 1.1012x over previous
import functools

import jax
import jax.numpy as jnp
from jax import lax
from jax.experimental import pallas as pl
from jax.experimental.pallas import tpu as pltpu
from jax.experimental.pallas import tpu_sc as plsc

_B, _L, _V = 16, 50, 100000
_N = _B * _L
_NPAD = 1024
_NW = 16
_PW = _NPAD // _NW
_WIN = 16


def _sc_loss(inp, tgt_pad, msk_pad):
    mesh = plsc.VectorSubcoreMesh(core_axis_name="c", subcore_axis_name="s",
                                  num_cores=1)

    @functools.partial(
        pl.kernel,
        mesh=mesh,
        out_type=jax.ShapeDtypeStruct((_NW + 1, 2, 16), jnp.float32),
        compiler_params=pltpu.CompilerParams(needs_layout_passes=False),
        scratch_types=[
            pltpu.VMEM((_PW,), jnp.int32),
            pltpu.VMEM((_PW,), jnp.float32),
            pltpu.VMEM((_PW, _WIN), jnp.float32),
            pltpu.VMEM((2, 16), jnp.float32),
            pltpu.VMEM((_NW, 2, 16), jnp.float32),
            pltpu.VMEM((16,), jnp.float32),
            pltpu.VMEM((16,), jnp.float32),
            pltpu.SemaphoreType.DMA,
        ],
    )
    def k(inp_hbm, tgt_hbm, msk_hbm, out_hbm, tgt_v, msk_v, win_v, part_v,
          red_v, bfly_v, fin_v, sem):
        c = lax.axis_index("c")
        s = lax.axis_index("s")

        @pl.when(c == 0)
        def _():
            base = pl.multiple_of(s * _PW, _PW)
            cp_t = pltpu.async_copy(tgt_hbm.at[pl.ds(base, _PW)], tgt_v, sem)
            cp_m = pltpu.async_copy(msk_hbm.at[pl.ds(base, _PW)], msk_v, sem)
            cp_t.wait()
            cp_m.wait()
            cps = []
            for i in range(_PW):
                if i % 16 == 0:
                    t16 = tgt_v[pl.ds(i, 16)]
                r = jnp.minimum(base + i, _N - 1)
                b = r // _L
                l = r % _L
                t = t16[i % 16]
                v0 = jnp.minimum((t // _WIN) * _WIN, _V - _WIN)
                cps.append(pltpu.async_copy(
                    inp_hbm.at[l, b, pl.ds(v0, _WIN)], win_v.at[i], sem))
            for cp in cps:
                cp.wait()
            num = jnp.zeros((16,), jnp.float32)
            den = jnp.zeros((16,), jnp.float32)
            for j in range(_PW // 16):
                t16 = tgt_v[pl.ds(j * 16, 16)]
                v016 = jnp.minimum((t16 // _WIN) * _WIN, _V - _WIN)
                off16 = t16 - v016
                rows16 = lax.iota(jnp.int32, 16) + (j * 16)
                g = plsc.load_gather(win_v, [rows16, off16])
                m = msk_v[pl.ds(j * 16, 16)]
                num = num + g * m
                den = den + m
            part_v[0] = num
            part_v[1] = den
            pltpu.sync_copy(part_v, out_hbm.at[s])

        plsc.subcore_barrier()

        @pl.when(jnp.logical_and(c == 0, s == 0))
        def _():
            pltpu.sync_copy(out_hbm.at[pl.ds(0, _NW)], red_v)
            num = jnp.zeros((16,), jnp.float32)
            den = jnp.zeros((16,), jnp.float32)
            for w in range(_NW):
                num = num + red_v[w, 0]
                den = den + red_v[w, 1]

            def lane_sum(vec):
                for sh in (8, 4, 2, 1):
                    bfly_v[...] = vec
                    ix = jnp.bitwise_and(lax.iota(jnp.int32, 16) + sh, 15)
                    vec = vec + plsc.load_gather(bfly_v, [ix])
                return vec

            num_t = lane_sum(num)
            den_t = lane_sum(den)
            fin_v[...] = -(num_t / den_t)
            pltpu.sync_copy(fin_v, out_hbm.at[_NW, 0])

    return k(inp, tgt_pad, msk_pad)


def kernel(input, target, mask):
    L = input.shape[1]
    target = target[:, :L]
    mask = mask[:, :L]
    tgt = target.reshape(-1).astype(jnp.int32)
    msk = mask.reshape(-1).astype(jnp.float32)
    tgt_pad = jnp.zeros((_NPAD,), jnp.int32).at[:_N].set(tgt)
    msk_pad = jnp.zeros((_NPAD,), jnp.float32).at[:_N].set(msk)
    inp_t = jnp.transpose(input, (1, 0, 2))
    out = _sc_loss(inp_t, tgt_pad, msk_pad)
    return out[_NW, 0, 0]
